# baseline (speedup 1.0000x reference)
import functools

import jax
import jax.numpy as jnp
from jax import lax
from jax.experimental import pallas as pl
from jax.experimental.pallas import tpu as pltpu
from jax.experimental.pallas import tpu_sc as plsc

B = 8
N = 2048
D = 256
L = 16
BIG_I32 = 1 << 30


def _nn_body(feats_hbm, coords_hbm, out_hbm, cbuf, idx_buf, f0row, dma_sem):
    cid = lax.axis_index("c")
    sid = lax.axis_index("s")

    @pl.when(sid < B // 2)
    def _():
        b = sid * 2 + cid
        pltpu.sync_copy(coords_hbm.at[b], cbuf)
        p0 = cbuf[0, pl.ds(0, L)]
        x0 = p0[0]
        y0 = p0[1]
        z0 = p0[2]
        lanes = lax.iota(jnp.int32, L)
        zero = jnp.zeros((L,), jnp.int32)

        UNROLL = 4
        SPAN = N // UNROLL

        def nn_step(j, carry):
            out = []
            for u in range(UNROLL):
                best_v, best_i = carry[u]
                off = u * SPAN + j * L
                flat = (off + lanes) * 3
                dx = plsc.load_gather(cbuf, [zero, flat]) - x0
                dy = plsc.load_gather(cbuf, [zero, flat + 1]) - y0
                dz = plsc.load_gather(cbuf, [zero, flat + 2]) - z0
                d2 = dx * dx + dy * dy + dz * dz
                pred = d2 < best_v
                out.append((jnp.where(pred, d2, best_v),
                            jnp.where(pred, off + lanes, best_i)))
            return tuple(out)

        init1 = (jnp.full((L,), BIG_I32, jnp.int32),
                 jnp.zeros((L,), jnp.int32))
        acc = lax.fori_loop(0, SPAN // L, nn_step, (init1,) * UNROLL)
        best_v, best_i = acc[0]
        for u in range(1, UNROLL):
            av, ai = acc[u]
            pred = av < best_v
            best_v = jnp.where(pred, av, best_v)
            best_i = jnp.where(pred, ai, best_i)
        bv_f = best_v.astype(jnp.float32)
        bi_f = best_i.astype(jnp.float32)
        mval = jnp.min(bv_f)
        nbr_f = jnp.min(jnp.where(bv_f == mval, bi_f, jnp.float32(1e9)))
        nbr = nbr_f.astype(jnp.int32)

        idx_buf[pl.ds(0, L)] = jnp.broadcast_to(b * N + nbr, (L,))
        pltpu.async_copy(feats_hbm.at[idx_buf], f0row, dma_sem).wait()
        pltpu.sync_copy(f0row.at[pl.ds(0, 1), :], out_hbm.at[b])


_nn_sc = functools.partial(
    pl.kernel,
    out_type=jax.ShapeDtypeStruct((B, 1, D), jnp.float32),
    mesh=plsc.VectorSubcoreMesh(core_axis_name="c", subcore_axis_name="s"),
    compiler_params=pltpu.CompilerParams(needs_layout_passes=False),
    scratch_types=[
        pltpu.VMEM((1, N * 3), jnp.int32),
        pltpu.VMEM((L,), jnp.int32),
        pltpu.VMEM((L, D), jnp.float32),
        pltpu.SemaphoreType.DMA,
    ],
)(_nn_body)


def _score_body(nbr_ref, feats_ref, o_ref):
    f = jnp.maximum(feats_ref[0], 0.0)
    rm = 1.0 / jnp.max(f, axis=1, keepdims=True)
    f0 = jnp.maximum(nbr_ref[0], 0.0)
    g = jnp.max(jnp.exp(f - f0) * (f * rm), axis=1, keepdims=True)
    s = jnp.sum(g * g)
    o_ref[0] = g * lax.rsqrt(s)


_score_tc = pl.pallas_call(
    _score_body,
    grid=(B,),
    in_specs=[
        pl.BlockSpec((1, 1, D), lambda b: (b, 0, 0)),
        pl.BlockSpec((1, N, D), lambda b: (b, 0, 0)),
    ],
    out_specs=pl.BlockSpec((1, N, 1), lambda b: (b, 0, 0)),
    out_shape=jax.ShapeDtypeStruct((B, N, 1), jnp.float32),
)


@jax.jit
def _run(coords, features):
    feats = features.reshape(B * N, D)
    coords_flat = coords.reshape(B, 1, N * 3)
    nbr_rows = _nn_sc(feats, coords_flat)
    score = _score_tc(nbr_rows, features)
    return score.reshape(B * N)


def kernel(coords, features, len_batch):
    del len_batch
    return _run(coords, features)


# speedup vs baseline: 1.2015x; 1.2015x over previous
import functools

import jax
import jax.numpy as jnp
from jax import lax
from jax.experimental import pallas as pl
from jax.experimental.pallas import tpu as pltpu
from jax.experimental.pallas import tpu_sc as plsc

B = 8
N = 2048
D = 256
L = 16
BIG_I32 = 1 << 30


def _nn_body(coords_hbm, out_hbm, cbuf, idx_buf):
    cid = lax.axis_index("c")
    sid = lax.axis_index("s")

    @pl.when(sid < B // 2)
    def _():
        b = sid * 2 + cid
        pltpu.sync_copy(coords_hbm.at[b], cbuf)
        x0 = cbuf[0, pl.ds(0, L)][0]
        y0 = cbuf[1, pl.ds(0, L)][0]
        z0 = cbuf[2, pl.ds(0, L)][0]
        lanes = lax.iota(jnp.int32, L)

        UNROLL = 4
        SPAN = N // UNROLL

        def nn_step(j, carry):
            out = []
            for u in range(UNROLL):
                best_v, best_i = carry[u]
                off = u * SPAN + j * L
                dx = cbuf[0, pl.ds(off, L)] - x0
                dy = cbuf[1, pl.ds(off, L)] - y0
                dz = cbuf[2, pl.ds(off, L)] - z0
                d2 = dx * dx + dy * dy + dz * dz
                pred = d2 < best_v
                out.append((jnp.where(pred, d2, best_v),
                            jnp.where(pred, off + lanes, best_i)))
            return tuple(out)

        init1 = (jnp.full((L,), BIG_I32, jnp.int32),
                 jnp.zeros((L,), jnp.int32))
        acc = lax.fori_loop(0, SPAN // L, nn_step, (init1,) * UNROLL)
        best_v, best_i = acc[0]
        for u in range(1, UNROLL):
            av, ai = acc[u]
            pred = av < best_v
            best_v = jnp.where(pred, av, best_v)
            best_i = jnp.where(pred, ai, best_i)
        bv_f = best_v.astype(jnp.float32)
        bi_f = best_i.astype(jnp.float32)
        mval = jnp.min(bv_f)
        nbr_f = jnp.min(jnp.where(bv_f == mval, bi_f, jnp.float32(1e9)))
        nbr = nbr_f.astype(jnp.int32)

        idx_buf[0, pl.ds(0, L)] = jnp.broadcast_to(nbr, (L,))
        pltpu.sync_copy(idx_buf, out_hbm.at[b])


_nn_sc = functools.partial(
    pl.kernel,
    out_type=jax.ShapeDtypeStruct((B, 1, L), jnp.int32),
    mesh=plsc.VectorSubcoreMesh(core_axis_name="c", subcore_axis_name="s"),
    compiler_params=pltpu.CompilerParams(needs_layout_passes=False),
    scratch_types=[
        pltpu.VMEM((3, N), jnp.int32),
        pltpu.VMEM((1, L), jnp.int32),
    ],
)(_nn_body)


def _score_body(nbr_ref, feats_ref, o_ref):
    nbr = nbr_ref[pl.program_id(0), 0, 0]
    f = jnp.maximum(feats_ref[0], 0.0)
    rm = 1.0 / jnp.max(f, axis=1, keepdims=True)
    f0 = jnp.maximum(feats_ref[0, pl.ds(nbr, 1), :], 0.0)
    g = jnp.max(jnp.exp(f - f0) * (f * rm), axis=1, keepdims=True)
    s = jnp.sum(g * g)
    o_ref[0] = g * lax.rsqrt(s)


_score_tc = pl.pallas_call(
    _score_body,
    grid=(B,),
    in_specs=[
        pl.BlockSpec(memory_space=pltpu.SMEM),
        pl.BlockSpec((1, N, D), lambda b: (b, 0, 0)),
    ],
    out_specs=pl.BlockSpec((1, N, 1), lambda b: (b, 0, 0)),
    out_shape=jax.ShapeDtypeStruct((B, N, 1), jnp.float32),
)


@jax.jit
def _run(coords, features):
    coords_t = coords.transpose(0, 2, 1)
    nbr_idx = _nn_sc(coords_t)
    score = _score_tc(nbr_idx, features)
    return score.reshape(B * N)


def kernel(coords, features, len_batch):
    del len_batch
    return _run(coords, features)


# speedup vs baseline: 1.2378x; 1.0302x over previous
import functools

import jax
import jax.numpy as jnp
from jax import lax
from jax.experimental import pallas as pl
from jax.experimental.pallas import tpu as pltpu
from jax.experimental.pallas import tpu_sc as plsc

B = 8
N = 2048
D = 256
L = 16
BIG_I32 = 1 << 30


def _nn_body(coords_hbm, out_hbm, cbuf, idx_buf):
    sid = lax.axis_index("s")

    @pl.when(sid < B)
    def _():
        b = sid
        pltpu.sync_copy(coords_hbm.at[b], cbuf)
        x0 = cbuf[0, pl.ds(0, L)][0]
        y0 = cbuf[1, pl.ds(0, L)][0]
        z0 = cbuf[2, pl.ds(0, L)][0]
        lanes = lax.iota(jnp.int32, L)

        UNROLL = 4
        SPAN = N // UNROLL

        def nn_step(j, carry):
            out = []
            for u in range(UNROLL):
                best_v, best_i = carry[u]
                off = u * SPAN + j * L
                dx = cbuf[0, pl.ds(off, L)] - x0
                dy = cbuf[1, pl.ds(off, L)] - y0
                dz = cbuf[2, pl.ds(off, L)] - z0
                d2 = dx * dx + dy * dy + dz * dz
                pred = d2 < best_v
                out.append((jnp.where(pred, d2, best_v),
                            jnp.where(pred, off + lanes, best_i)))
            return tuple(out)

        init1 = (jnp.full((L,), BIG_I32, jnp.int32),
                 jnp.zeros((L,), jnp.int32))
        acc = lax.fori_loop(0, SPAN // L, nn_step, (init1,) * UNROLL)
        best_v, best_i = acc[0]
        for u in range(1, UNROLL):
            av, ai = acc[u]
            pred = av < best_v
            best_v = jnp.where(pred, av, best_v)
            best_i = jnp.where(pred, ai, best_i)
        bv_f = best_v.astype(jnp.float32)
        bi_f = best_i.astype(jnp.float32)
        mval = jnp.min(bv_f)
        nbr_f = jnp.min(jnp.where(bv_f == mval, bi_f, jnp.float32(1e9)))
        nbr = nbr_f.astype(jnp.int32)

        idx_buf[0, pl.ds(0, L)] = jnp.broadcast_to(nbr, (L,))
        pltpu.sync_copy(idx_buf, out_hbm.at[b])


_nn_sc = functools.partial(
    pl.kernel,
    out_type=jax.ShapeDtypeStruct((B, 1, L), jnp.int32),
    mesh=plsc.VectorSubcoreMesh(core_axis_name="c", subcore_axis_name="s",
                                num_cores=1),
    compiler_params=pltpu.CompilerParams(needs_layout_passes=False),
    scratch_types=[
        pltpu.VMEM((3, N), jnp.int32),
        pltpu.VMEM((1, L), jnp.int32),
    ],
)(_nn_body)


def _score_body(nbr_ref, feats_ref, o_ref):
    nbr = nbr_ref[pl.program_id(0), 0, 0]
    f = jnp.maximum(feats_ref[0], 0.0)
    rm = 1.0 / jnp.max(f, axis=1, keepdims=True)
    f0 = jnp.maximum(feats_ref[0, pl.ds(nbr, 1), :], 0.0)
    g = jnp.max(jnp.exp(f - f0) * (f * rm), axis=1, keepdims=True)
    s = jnp.sum(g * g)
    o_ref[0] = g * lax.rsqrt(s)


_score_tc = pl.pallas_call(
    _score_body,
    grid=(B,),
    in_specs=[
        pl.BlockSpec(memory_space=pltpu.SMEM),
        pl.BlockSpec((1, N, D), lambda b: (b, 0, 0)),
    ],
    out_specs=pl.BlockSpec((1, N, 1), lambda b: (b, 0, 0)),
    out_shape=jax.ShapeDtypeStruct((B, N, 1), jnp.float32),
)


@jax.jit
def _run(coords, features):
    coords_t = coords.transpose(0, 2, 1)
    nbr_idx = _nn_sc(coords_t)
    score = _score_tc(nbr_idx, features)
    return score.reshape(B * N)


def kernel(coords, features, len_batch):
    del len_batch
    return _run(coords, features)


# speedup vs baseline: 1.3619x; 1.1003x over previous
import functools

import jax
import jax.numpy as jnp
from jax import lax
from jax.experimental import pallas as pl
from jax.experimental.pallas import tpu as pltpu
from jax.experimental.pallas import tpu_sc as plsc

B = 8
N = 2048
D = 256
L = 16
BIG_I32 = 1 << 30


def _nn_body(coords_hbm, out_hbm, cbuf, idx_buf):
    sid = lax.axis_index("s")

    @pl.when(sid < B)
    def _():
        b = sid
        pltpu.sync_copy(coords_hbm.at[b], cbuf)
        x0 = cbuf[0, pl.ds(0, L)][0]
        y0 = cbuf[1, pl.ds(0, L)][0]
        z0 = cbuf[2, pl.ds(0, L)][0]
        lanes = lax.iota(jnp.int32, L)

        UNROLL = 4
        SPAN = N // UNROLL

        def nn_step(j, carry):
            out = []
            for u in range(UNROLL):
                best_v, best_i = carry[u]
                off = u * SPAN + j * L
                dx = cbuf[0, pl.ds(off, L)] - x0
                dy = cbuf[1, pl.ds(off, L)] - y0
                dz = cbuf[2, pl.ds(off, L)] - z0
                d2 = dx * dx + dy * dy + dz * dz
                pred = d2 < best_v
                out.append((jnp.where(pred, d2, best_v),
                            jnp.where(pred, off + lanes, best_i)))
            return tuple(out)

        init1 = (jnp.full((L,), BIG_I32, jnp.int32),
                 jnp.zeros((L,), jnp.int32))
        acc = lax.fori_loop(0, SPAN // L, nn_step, (init1,) * UNROLL)
        best_v, best_i = acc[0]
        for u in range(1, UNROLL):
            av, ai = acc[u]
            pred = av < best_v
            best_v = jnp.where(pred, av, best_v)
            best_i = jnp.where(pred, ai, best_i)
        bv_f = best_v.astype(jnp.float32)
        bi_f = best_i.astype(jnp.float32)
        mval = jnp.min(bv_f)
        nbr_f = jnp.min(jnp.where(bv_f == mval, bi_f, jnp.float32(1e9)))
        nbr = nbr_f.astype(jnp.int32)

        idx_buf[0, pl.ds(0, L)] = jnp.broadcast_to(nbr, (L,))
        pltpu.sync_copy(idx_buf, out_hbm.at[b])


_nn_sc = functools.partial(
    pl.kernel,
    out_type=jax.ShapeDtypeStruct((B, 1, L), jnp.int32),
    mesh=plsc.VectorSubcoreMesh(core_axis_name="c", subcore_axis_name="s",
                                num_cores=1),
    compiler_params=pltpu.CompilerParams(needs_layout_passes=False),
    scratch_types=[
        pltpu.VMEM((3, N), jnp.int32),
        pltpu.VMEM((1, L), jnp.int32),
    ],
)(_nn_body)


def _score_body(nbr_ref, feats_ref, o_ref):
    nbr = nbr_ref[pl.program_id(0), 0, 0]
    f = jnp.maximum(feats_ref[0], 0.0)
    rm = 1.0 / jnp.max(f, axis=1, keepdims=True)
    f0 = jnp.maximum(feats_ref[0, pl.ds(nbr, 1), :], 0.0)
    g = jnp.max(jnp.exp(f - f0) * (f * rm), axis=1)
    s = jnp.sum(g * g)
    o_ref[...] = g * lax.rsqrt(s)


_score_tc = pl.pallas_call(
    _score_body,
    grid=(B,),
    in_specs=[
        pl.BlockSpec(memory_space=pltpu.SMEM),
        pl.BlockSpec((1, N, D), lambda b: (b, 0, 0)),
    ],
    out_specs=pl.BlockSpec((N,), lambda b: (b,)),
    out_shape=jax.ShapeDtypeStruct((B * N,), jnp.float32),
)


@jax.jit
def _run(coords, features):
    coords_t = coords.transpose(0, 2, 1)
    nbr_idx = _nn_sc(coords_t)
    return _score_tc(nbr_idx, features)


def kernel(coords, features, len_batch):
    del len_batch
    return _run(coords, features)
